# baseline (speedup 1.0000x reference)
import functools

import jax
import jax.numpy as jnp
from jax import lax
from jax.experimental import pallas as pl
from jax.experimental.pallas import tpu as pltpu
from jax.experimental.pallas import tpu_sc as plsc

N = 50000
E = 800000
B = 4
F_IN = 32
F_OUT = 32
K = 5

NUM_TILES = 16
SLICE = 3136
NPAD = NUM_TILES * SLICE
LANES = 16

CHUNK_ROWS = 10
CHUNK = CHUNK_ROWS * 128
NCHUNK = 40
ROWS_PER_TILE = NCHUNK * CHUNK_ROWS
EROWS = NUM_TILES * ROWS_PER_TILE
E_PAD = EROWS * 128

KROWS = 8

TC_NB = 7168
TC_STEPS = NPAD // TC_NB


def _sc_taps_body(row_hbm, col_hbm, w_hbm, out_hbm,
                  u_sh, v_vmem, row_b0, col_b0, w_b0, ctr_b0,
                  row_b1, col_b1, w_b1, ctr_b1, zbuf, tmp_buf,
                  sem_v, sem_load, sem_col, sem_scat):
    cid = lax.axis_index("c")
    sid = lax.axis_index("s")
    is_w = cid == 0
    off = sid * SLICE
    row_bufs = (row_b0, row_b1)
    col_bufs = (col_b0, col_b1)
    w_bufs = (w_b0, w_b1)
    ctr_bufs = (ctr_b0, ctr_b1)

    def prefetch(j, b):
        base = sid * ROWS_PER_TILE + j * CHUNK_ROWS
        sl = pl.ds(base, CHUNK_ROWS)
        pltpu.async_copy(row_hbm.at[sl], row_bufs[b], sem_load.at[b])
        pltpu.async_copy(w_hbm.at[sl], w_bufs[b], sem_load.at[b])

    def drain_load(b):
        sl = pl.ds(0, CHUNK_ROWS)
        pltpu.make_async_copy(row_hbm.at[sl], row_bufs[b],
                              sem_load.at[b]).wait()
        pltpu.make_async_copy(w_hbm.at[sl], w_bufs[b], sem_load.at[b]).wait()

    def prefetch_col(j, b):
        base = sid * ROWS_PER_TILE + j * CHUNK_ROWS
        pltpu.async_copy(col_hbm.at[pl.ds(base, CHUNK_ROWS)], col_bufs[b],
                         sem_col.at[b])

    def drain_col(b):
        pltpu.make_async_copy(col_hbm.at[pl.ds(0, CHUNK_ROWS)], col_bufs[b],
                              sem_col.at[b]).wait()

    def issue_scat(b):
        for r in range(CHUNK_ROWS):
            pltpu.async_copy(ctr_bufs[b].at[r],
                             u_sh.at[col_bufs[b].at[r]],
                             sem_scat.at[b], add=True)

    def drain_scat(b):
        for r in range(CHUNK_ROWS):
            pltpu.make_async_copy(ctr_bufs[b].at[r],
                                  u_sh.at[col_bufs[b].at[r]],
                                  sem_scat.at[b]).wait()

    def compute(b):
        for r in range(CHUNK_ROWS):
            for c in range(128 // LANES):
                sl = pl.ds(c * LANES, LANES)
                idx = row_bufs[b][r, sl]
                vals = plsc.load_gather(v_vmem, [idx])
                ctr_bufs[b][r, sl] = vals * w_bufs[b][r, sl]

    @pl.when(is_w)
    def _init():
        for i in range(SLICE // LANES):
            zbuf[pl.ds(i * LANES, LANES)] = jnp.zeros((LANES,), jnp.float32)
        for r in (0, 5, 6, 7):
            pltpu.sync_copy(zbuf, out_hbm.at[pl.ds(r * NPAD + off, SLICE)])
        pltpu.sync_copy(zbuf, u_sh.at[pl.ds(off, SLICE)])

    @pl.when(is_w & (sid == 0))
    def _onehot():
        iot = lax.iota(jnp.int32, LANES)
        ctr_b0[0, pl.ds(0, LANES)] = jnp.where(
            iot == 0, jnp.float32(1.0), jnp.float32(0.0))
        pltpu.sync_copy(ctr_b0.at[0, pl.ds(0, LANES)],
                        out_hbm.at[pl.ds(0, LANES)])

    plsc.subcore_barrier()

    for k in range(K - 1):
        @pl.when(is_w)
        def _tap():
            vcp = pltpu.async_copy(out_hbm.at[pl.ds(k * NPAD, NPAD)],
                                   v_vmem, sem_v)
            prefetch(0, 0)
            vcp.wait()

            def pair_body(jj, carry):
                for b in range(2):
                    j = 2 * jj + b
                    drain_load(b)
                    if b == 0:
                        prefetch(j + 1, 1)
                    else:
                        @pl.when(jj <= NCHUNK // 2 - 2)
                        def _p():
                            prefetch(j + 1, 0)
                    @pl.when(jj >= 1)
                    def _d():
                        drain_scat(b)
                    prefetch_col(j, b)
                    compute(b)
                    drain_col(b)
                    issue_scat(b)
                return carry

            lax.fori_loop(0, NCHUNK // 2, pair_body, 0)
            drain_scat(0)
            drain_scat(1)

        plsc.subcore_barrier()

        @pl.when(is_w)
        def _flush():
            pltpu.sync_copy(u_sh.at[pl.ds(off, SLICE)], tmp_buf)
            pltpu.sync_copy(
                tmp_buf, out_hbm.at[pl.ds((k + 1) * NPAD + off, SLICE)])
            pltpu.sync_copy(zbuf, u_sh.at[pl.ds(off, SLICE)])

        plsc.subcore_barrier()


_sc_taps = functools.partial(
    pl.kernel,
    out_type=jax.ShapeDtypeStruct((KROWS * NPAD,), jnp.float32),
    mesh=plsc.VectorSubcoreMesh(core_axis_name="c", subcore_axis_name="s"),
    compiler_params=pltpu.CompilerParams(use_tc_tiling_on_sc=False,
                                         needs_layout_passes=False),
    scratch_types=[
        pltpu.VMEM_SHARED((NPAD,), jnp.float32),
        pltpu.VMEM((NPAD,), jnp.float32),
        pltpu.VMEM((CHUNK_ROWS, 128), jnp.int32),
        pltpu.VMEM((CHUNK_ROWS, 128), jnp.int32),
        pltpu.VMEM((CHUNK_ROWS, 128), jnp.float32),
        pltpu.VMEM((CHUNK_ROWS, 128), jnp.float32),
        pltpu.VMEM((CHUNK_ROWS, 128), jnp.int32),
        pltpu.VMEM((CHUNK_ROWS, 128), jnp.int32),
        pltpu.VMEM((CHUNK_ROWS, 128), jnp.float32),
        pltpu.VMEM((CHUNK_ROWS, 128), jnp.float32),
        pltpu.VMEM((SLICE,), jnp.float32),
        pltpu.VMEM((SLICE,), jnp.float32),
        pltpu.SemaphoreType.DMA,
        pltpu.SemaphoreType.DMA((2,)),
        pltpu.SemaphoreType.DMA((2,)),
        pltpu.SemaphoreType.DMA((2,)),
    ],
)(_sc_taps_body)


def _tc_reduce_body(v_hbm, x_ref, h_ref, b_ref, out_ref, acc_ref, v_vmem, sem):
    i = pl.program_id(0)

    @pl.when(i == 0)
    def _zero():
        acc_ref[...] = jnp.zeros_like(acc_ref)
        pltpu.make_async_copy(v_hbm, v_vmem, sem).start()
        pltpu.make_async_copy(v_hbm, v_vmem, sem).wait()

    vb = v_vmem[:, pl.ds(i * TC_NB, TC_NB)]
    xb = x_ref[...]
    valid = N - i * TC_NB
    node_pos = lax.broadcasted_iota(jnp.int32, (B, TC_NB, F_IN), 1)
    xb = jnp.where(node_pos < valid, xb, jnp.float32(0.0))
    for bb in range(B):
        acc_ref[bb] += jnp.dot(vb, xb[bb],
                               preferred_element_type=jnp.float32)

    @pl.when(i == TC_STEPS - 1)
    def _epi():
        acc = acc_ref[...]
        h = h_ref[...]
        out = b_ref[...]
        for kk in range(K):
            out = out + jnp.dot(acc[:, kk, :], h[kk],
                                preferred_element_type=jnp.float32)
        out_ref[...] = out


def _tc_reduce(v, x, h2, b2):
    return pl.pallas_call(
        _tc_reduce_body,
        grid=(TC_STEPS,),
        in_specs=[
            pl.BlockSpec(memory_space=pl.ANY),
            pl.BlockSpec((B, TC_NB, F_IN), lambda i: (0, i, 0)),
            pl.BlockSpec((KROWS, F_IN, F_OUT), lambda i: (0, 0, 0)),
            pl.BlockSpec((1, F_OUT), lambda i: (0, 0)),
        ],
        out_specs=pl.BlockSpec((B, F_OUT), lambda i: (0, 0)),
        out_shape=jax.ShapeDtypeStruct((B, F_OUT), jnp.float32),
        scratch_shapes=[
            pltpu.VMEM((B, KROWS, F_IN), jnp.float32),
            pltpu.VMEM((KROWS, NPAD), jnp.float32),
            pltpu.SemaphoreType.DMA,
        ],
    )(v, x, h2, b2)


def kernel(x, edge_row, edge_col, edge_w, H, b):
    er = edge_row.reshape(E)
    ec = edge_col.reshape(E)
    ew = edge_w.reshape(E)
    npad = E_PAD - E
    pad_idx = (jnp.arange(npad, dtype=jnp.int32) % N)
    row_p = jnp.concatenate([er, pad_idx]).reshape(EROWS, 128)
    col_p = jnp.concatenate([ec, pad_idx]).reshape(EROWS, 128)
    w_p = jnp.concatenate([ew, jnp.zeros((npad,), jnp.float32)]
                          ).reshape(EROWS, 128)

    v = _sc_taps(row_p, col_p, w_p).reshape(KROWS, NPAD)

    h2 = jnp.zeros((KROWS, F_IN, F_OUT), jnp.float32).at[:K].set(H[0])
    return _tc_reduce(v, x, h2, b.reshape(1, F_OUT))


# speedup vs baseline: 1.0086x; 1.0086x over previous
import functools

import jax
import jax.numpy as jnp
from jax import lax
from jax.experimental import pallas as pl
from jax.experimental.pallas import tpu as pltpu
from jax.experimental.pallas import tpu_sc as plsc

N = 50000
E = 800000
B = 4
F_IN = 32
F_OUT = 32
K = 5

NUM_TILES = 16
SLICE = 3136
NPAD = NUM_TILES * SLICE
LANES = 16

CHUNK_ROWS = 10
CHUNK = CHUNK_ROWS * 128
NCHUNK = 39
ROWS_PER_TILE = NCHUNK * CHUNK_ROWS
EROWS = E // 128
TAIL_ROW0 = NUM_TILES * ROWS_PER_TILE

KROWS = 8

TC_NB = 7168
TC_STEPS = NPAD // TC_NB


def _sc_taps_body(row_hbm, col_hbm, w_hbm, out_hbm,
                  u_sh, v_vmem, row_b0, col_b0, w_b0, ctr_b0,
                  row_b1, col_b1, w_b1, ctr_b1, zbuf, tmp_buf,
                  sem_v, sem_load, sem_col, sem_scat):
    cid = lax.axis_index("c")
    sid = lax.axis_index("s")
    is_w = cid == 0
    off = sid * SLICE
    row_bufs = (row_b0, row_b1)
    col_bufs = (col_b0, col_b1)
    w_bufs = (w_b0, w_b1)
    ctr_bufs = (ctr_b0, ctr_b1)

    def prefetch(j, b):
        base = sid * ROWS_PER_TILE + j * CHUNK_ROWS
        sl = pl.ds(base, CHUNK_ROWS)
        pltpu.async_copy(row_hbm.at[sl], row_bufs[b], sem_load.at[b])
        pltpu.async_copy(w_hbm.at[sl], w_bufs[b], sem_load.at[b])

    def drain_load(b):
        sl = pl.ds(0, CHUNK_ROWS)
        pltpu.make_async_copy(row_hbm.at[sl], row_bufs[b],
                              sem_load.at[b]).wait()
        pltpu.make_async_copy(w_hbm.at[sl], w_bufs[b], sem_load.at[b]).wait()

    def prefetch_col(j, b):
        base = sid * ROWS_PER_TILE + j * CHUNK_ROWS
        pltpu.async_copy(col_hbm.at[pl.ds(base, CHUNK_ROWS)], col_bufs[b],
                         sem_col.at[b])

    def drain_col(b):
        pltpu.make_async_copy(col_hbm.at[pl.ds(0, CHUNK_ROWS)], col_bufs[b],
                              sem_col.at[b]).wait()

    def issue_scat(b):
        for r in range(CHUNK_ROWS):
            pltpu.async_copy(ctr_bufs[b].at[r],
                             u_sh.at[col_bufs[b].at[r]],
                             sem_scat.at[b], add=True)

    def drain_scat(b):
        for r in range(CHUNK_ROWS):
            pltpu.make_async_copy(ctr_bufs[b].at[r],
                                  u_sh.at[col_bufs[b].at[r]],
                                  sem_scat.at[b]).wait()

    def compute(b, nrows=CHUNK_ROWS):
        for r in range(nrows):
            for c in range(128 // LANES):
                sl = pl.ds(c * LANES, LANES)
                idx = row_bufs[b][r, sl]
                vals = plsc.load_gather(v_vmem, [idx])
                ctr_bufs[b][r, sl] = vals * w_bufs[b][r, sl]

    @pl.when(is_w)
    def _init():
        for i in range(SLICE // LANES):
            zbuf[pl.ds(i * LANES, LANES)] = jnp.zeros((LANES,), jnp.float32)
        for r in (0, 5, 6, 7):
            pltpu.sync_copy(zbuf, out_hbm.at[pl.ds(r * NPAD + off, SLICE)])
        pltpu.sync_copy(zbuf, u_sh.at[pl.ds(off, SLICE)])

    @pl.when(is_w & (sid == 0))
    def _onehot():
        iot = lax.iota(jnp.int32, LANES)
        ctr_b0[0, pl.ds(0, LANES)] = jnp.where(
            iot == 0, jnp.float32(1.0), jnp.float32(0.0))
        pltpu.sync_copy(ctr_b0.at[0, pl.ds(0, LANES)],
                        out_hbm.at[pl.ds(0, LANES)])

    plsc.subcore_barrier()

    for k in range(K - 1):
        @pl.when(is_w)
        def _tap():
            vcp = pltpu.async_copy(out_hbm.at[pl.ds(k * NPAD, NPAD)],
                                   v_vmem, sem_v)
            prefetch(0, 0)
            vcp.wait()

            def pair_body(jj, carry):
                for b in range(2):
                    j = 2 * jj + b
                    drain_load(b)
                    prefetch(j + 1, 1 - b)
                    @pl.when(jj >= 1)
                    def _d():
                        drain_scat(b)
                    prefetch_col(j, b)
                    compute(b)
                    drain_col(b)
                    issue_scat(b)
                return carry

            lax.fori_loop(0, (NCHUNK - 1) // 2, pair_body, 0)

            drain_load(0)
            drain_scat(0)
            prefetch_col(NCHUNK - 1, 0)
            compute(0)
            drain_col(0)
            issue_scat(0)
            drain_scat(1)
            @pl.when(sid < EROWS - TAIL_ROW0)
            def _tail():
                trow = TAIL_ROW0 + sid
                tsl = pl.ds(trow, 1)
                dsl = pl.ds(0, 1)
                pltpu.sync_copy(row_hbm.at[tsl], row_bufs[1].at[dsl])
                pltpu.sync_copy(col_hbm.at[tsl], col_bufs[1].at[dsl])
                pltpu.sync_copy(w_hbm.at[tsl], w_bufs[1].at[dsl])
                compute(1, nrows=1)
                pltpu.sync_copy(ctr_bufs[1].at[0],
                                u_sh.at[col_bufs[1].at[0]], add=True)
            drain_scat(0)

        plsc.subcore_barrier()

        @pl.when(is_w)
        def _flush():
            pltpu.sync_copy(u_sh.at[pl.ds(off, SLICE)], tmp_buf)
            pltpu.sync_copy(
                tmp_buf, out_hbm.at[pl.ds((k + 1) * NPAD + off, SLICE)])
            pltpu.sync_copy(zbuf, u_sh.at[pl.ds(off, SLICE)])

        plsc.subcore_barrier()


_sc_taps = functools.partial(
    pl.kernel,
    out_type=jax.ShapeDtypeStruct((KROWS * NPAD,), jnp.float32),
    mesh=plsc.VectorSubcoreMesh(core_axis_name="c", subcore_axis_name="s"),
    compiler_params=pltpu.CompilerParams(use_tc_tiling_on_sc=False,
                                         needs_layout_passes=False),
    scratch_types=[
        pltpu.VMEM_SHARED((NPAD,), jnp.float32),
        pltpu.VMEM((NPAD,), jnp.float32),
        pltpu.VMEM((CHUNK_ROWS, 128), jnp.int32),
        pltpu.VMEM((CHUNK_ROWS, 128), jnp.int32),
        pltpu.VMEM((CHUNK_ROWS, 128), jnp.float32),
        pltpu.VMEM((CHUNK_ROWS, 128), jnp.float32),
        pltpu.VMEM((CHUNK_ROWS, 128), jnp.int32),
        pltpu.VMEM((CHUNK_ROWS, 128), jnp.int32),
        pltpu.VMEM((CHUNK_ROWS, 128), jnp.float32),
        pltpu.VMEM((CHUNK_ROWS, 128), jnp.float32),
        pltpu.VMEM((SLICE,), jnp.float32),
        pltpu.VMEM((SLICE,), jnp.float32),
        pltpu.SemaphoreType.DMA,
        pltpu.SemaphoreType.DMA((2,)),
        pltpu.SemaphoreType.DMA((2,)),
        pltpu.SemaphoreType.DMA((2,)),
    ],
)(_sc_taps_body)


def _tc_reduce_body(v_hbm, x_ref, h_ref, b_ref, out_ref, acc_ref, v_vmem, sem):
    i = pl.program_id(0)

    @pl.when(i == 0)
    def _zero():
        acc_ref[...] = jnp.zeros_like(acc_ref)
        pltpu.make_async_copy(v_hbm, v_vmem, sem).start()
        pltpu.make_async_copy(v_hbm, v_vmem, sem).wait()

    vb = v_vmem[:, pl.ds(i * TC_NB, TC_NB)]
    xb = x_ref[...]
    valid = N - i * TC_NB
    node_pos = lax.broadcasted_iota(jnp.int32, (B, TC_NB, F_IN), 1)
    xb = jnp.where(node_pos < valid, xb, jnp.float32(0.0))
    for bb in range(B):
        acc_ref[bb] += jnp.dot(vb, xb[bb],
                               preferred_element_type=jnp.float32)

    @pl.when(i == TC_STEPS - 1)
    def _epi():
        acc = acc_ref[...]
        h = h_ref[...]
        out = b_ref[...]
        for kk in range(K):
            out = out + jnp.dot(acc[:, kk, :], h[kk],
                                preferred_element_type=jnp.float32)
        out_ref[...] = out


def _tc_reduce(v, x, h2, b2):
    return pl.pallas_call(
        _tc_reduce_body,
        grid=(TC_STEPS,),
        in_specs=[
            pl.BlockSpec(memory_space=pl.ANY),
            pl.BlockSpec((B, TC_NB, F_IN), lambda i: (0, i, 0)),
            pl.BlockSpec((KROWS, F_IN, F_OUT), lambda i: (0, 0, 0)),
            pl.BlockSpec((1, F_OUT), lambda i: (0, 0)),
        ],
        out_specs=pl.BlockSpec((B, F_OUT), lambda i: (0, 0)),
        out_shape=jax.ShapeDtypeStruct((B, F_OUT), jnp.float32),
        scratch_shapes=[
            pltpu.VMEM((B, KROWS, F_IN), jnp.float32),
            pltpu.VMEM((KROWS, NPAD), jnp.float32),
            pltpu.SemaphoreType.DMA,
        ],
    )(v, x, h2, b2)


def kernel(x, edge_row, edge_col, edge_w, H, b):
    row_p = edge_row.reshape(EROWS, 128)
    col_p = edge_col.reshape(EROWS, 128)
    w_p = edge_w.reshape(EROWS, 128)

    v = _sc_taps(row_p, col_p, w_p).reshape(KROWS, NPAD)

    h2 = jnp.zeros((KROWS, F_IN, F_OUT), jnp.float32).at[:K].set(H[0])
    return _tc_reduce(v, x, h2, b.reshape(1, F_OUT))


# speedup vs baseline: 1.1628x; 1.1529x over previous
import functools

import jax
import jax.numpy as jnp
from jax import lax
from jax.experimental import pallas as pl
from jax.experimental.pallas import tpu as pltpu
from jax.experimental.pallas import tpu_sc as plsc

N = 50000
E = 800000
B = 4
F_IN = 32
F_OUT = 32
K = 5

NUM_TILES = 16
SLICE = 3136
NPAD = NUM_TILES * SLICE
LANES = 16

CHUNK_ROWS = 10
CHUNK = CHUNK_ROWS * 128
NCHUNK = 39
ROWS_PER_TILE = NCHUNK * CHUNK_ROWS
EROWS = E // 128
TAIL_ROW0 = NUM_TILES * ROWS_PER_TILE

KROWS = 8

TC_NB = 7168
TC_STEPS = NPAD // TC_NB


def _sc_taps_body(row_hbm, col_hbm, w_hbm, out_hbm,
                  u_sh, v_vmem, row_b0, col_b0, w_b0, ctr_b0,
                  row_b1, col_b1, w_b1, ctr_b1, zbuf, tmp_buf,
                  sem_v, sem_load, sem_col, sem_scat):
    cid = lax.axis_index("c")
    sid = lax.axis_index("s")
    is_w = cid == 0
    off = sid * SLICE
    row_bufs = (row_b0, row_b1)
    col_bufs = (col_b0, col_b1)
    w_bufs = (w_b0, w_b1)
    ctr_bufs = (ctr_b0, ctr_b1)

    def prefetch(j, b):
        base = sid * ROWS_PER_TILE + j * CHUNK_ROWS
        sl = pl.ds(base, CHUNK_ROWS)
        pltpu.async_copy(row_hbm.at[sl], row_bufs[b], sem_load.at[b])
        pltpu.async_copy(w_hbm.at[sl], w_bufs[b], sem_load.at[b])

    def drain_load(b):
        sl = pl.ds(0, CHUNK_ROWS)
        pltpu.make_async_copy(row_hbm.at[sl], row_bufs[b],
                              sem_load.at[b]).wait()
        pltpu.make_async_copy(w_hbm.at[sl], w_bufs[b], sem_load.at[b]).wait()

    def prefetch_col(j, b):
        base = sid * ROWS_PER_TILE + j * CHUNK_ROWS
        pltpu.async_copy(col_hbm.at[pl.ds(base, CHUNK_ROWS)], col_bufs[b],
                         sem_col.at[b])

    def drain_col(b):
        pltpu.make_async_copy(col_hbm.at[pl.ds(0, CHUNK_ROWS)], col_bufs[b],
                              sem_col.at[b]).wait()

    def issue_scat(b):
        for r in range(CHUNK_ROWS):
            pltpu.async_copy(ctr_bufs[b].at[r],
                             u_sh.at[col_bufs[b].at[r]],
                             sem_scat.at[b], add=True)

    def drain_scat(b):
        for r in range(CHUNK_ROWS):
            pltpu.make_async_copy(ctr_bufs[b].at[r],
                                  u_sh.at[col_bufs[b].at[r]],
                                  sem_scat.at[b]).wait()

    def compute(b, nrows=CHUNK_ROWS):
        for r in range(nrows):
            for c in range(128 // LANES):
                sl = pl.ds(c * LANES, LANES)
                idx = row_bufs[b][r, sl]
                vals = plsc.load_gather(v_vmem, [idx])
                ctr_bufs[b][r, sl] = vals * w_bufs[b][r, sl]

    @pl.when(is_w)
    def _init():
        for i in range(SLICE // LANES):
            zbuf[pl.ds(i * LANES, LANES)] = jnp.zeros((LANES,), jnp.float32)
        for r in (0, 5, 6, 7):
            pltpu.sync_copy(zbuf, out_hbm.at[pl.ds(r * NPAD + off, SLICE)])
        pltpu.sync_copy(zbuf, u_sh.at[pl.ds(off, SLICE)])

    @pl.when(is_w & (sid == 0))
    def _onehot():
        iot = lax.iota(jnp.int32, LANES)
        ctr_b0[0, pl.ds(0, LANES)] = jnp.where(
            iot == 0, jnp.float32(1.0), jnp.float32(0.0))
        pltpu.sync_copy(ctr_b0.at[0, pl.ds(0, LANES)],
                        out_hbm.at[pl.ds(0, LANES)])

    plsc.subcore_barrier()

    for k in range(K - 1):
        @pl.when(is_w)
        def _tap():
            vcp = pltpu.async_copy(out_hbm.at[pl.ds(k * NPAD, NPAD)],
                                   v_vmem, sem_v)
            prefetch(0, 0)
            vcp.wait()

            def pair_body(jj, carry):
                for b in range(2):
                    j = 2 * jj + b
                    drain_load(b)
                    prefetch(j + 1, 1 - b)
                    @pl.when(jj >= 1)
                    def _d():
                        drain_scat(b)
                    prefetch_col(j, b)
                    compute(b)
                    drain_col(b)
                    issue_scat(b)
                return carry

            lax.fori_loop(0, (NCHUNK - 1) // 2, pair_body, 0)

            drain_load(0)
            drain_scat(0)
            prefetch_col(NCHUNK - 1, 0)
            compute(0)
            drain_col(0)
            issue_scat(0)
            drain_scat(1)
            @pl.when(sid < EROWS - TAIL_ROW0)
            def _tail():
                trow = TAIL_ROW0 + sid
                tsl = pl.ds(trow, 1)
                dsl = pl.ds(0, 1)
                pltpu.sync_copy(row_hbm.at[tsl], row_bufs[1].at[dsl])
                pltpu.sync_copy(col_hbm.at[tsl], col_bufs[1].at[dsl])
                pltpu.sync_copy(w_hbm.at[tsl], w_bufs[1].at[dsl])
                compute(1, nrows=1)
                pltpu.sync_copy(ctr_bufs[1].at[0],
                                u_sh.at[col_bufs[1].at[0]], add=True)
            drain_scat(0)

        plsc.subcore_barrier()

        @pl.when(is_w)
        def _flush():
            pltpu.sync_copy(u_sh.at[pl.ds(off, SLICE)], tmp_buf)
            pltpu.sync_copy(
                tmp_buf, out_hbm.at[pl.ds((k + 1) * NPAD + off, SLICE)])
            pltpu.sync_copy(zbuf, u_sh.at[pl.ds(off, SLICE)])

        plsc.subcore_barrier()


_sc_taps = functools.partial(
    pl.kernel,
    out_type=jax.ShapeDtypeStruct((KROWS * NPAD,), jnp.float32),
    mesh=plsc.VectorSubcoreMesh(core_axis_name="c", subcore_axis_name="s"),
    compiler_params=pltpu.CompilerParams(use_tc_tiling_on_sc=False,
                                         needs_layout_passes=False),
    scratch_types=[
        pltpu.VMEM_SHARED((NPAD,), jnp.float32),
        pltpu.VMEM((NPAD,), jnp.float32),
        pltpu.VMEM((CHUNK_ROWS, 128), jnp.int32),
        pltpu.VMEM((CHUNK_ROWS, 128), jnp.int32),
        pltpu.VMEM((CHUNK_ROWS, 128), jnp.float32),
        pltpu.VMEM((CHUNK_ROWS, 128), jnp.float32),
        pltpu.VMEM((CHUNK_ROWS, 128), jnp.int32),
        pltpu.VMEM((CHUNK_ROWS, 128), jnp.int32),
        pltpu.VMEM((CHUNK_ROWS, 128), jnp.float32),
        pltpu.VMEM((CHUNK_ROWS, 128), jnp.float32),
        pltpu.VMEM((SLICE,), jnp.float32),
        pltpu.VMEM((SLICE,), jnp.float32),
        pltpu.SemaphoreType.DMA,
        pltpu.SemaphoreType.DMA((2,)),
        pltpu.SemaphoreType.DMA((2,)),
        pltpu.SemaphoreType.DMA((2,)),
    ],
)(_sc_taps_body)


def _tc_reduce_body(v_hbm, x_ref, h_ref, b_ref, out_ref, acc_ref, v_vmem, sem):
    i = pl.program_id(0)

    @pl.when(i == 0)
    def _zero():
        acc_ref[...] = jnp.zeros_like(acc_ref)
        pltpu.make_async_copy(v_hbm, v_vmem, sem).start()
        pltpu.make_async_copy(v_hbm, v_vmem, sem).wait()

    vb = v_vmem[:, pl.ds(i * TC_NB, TC_NB)]
    xb = x_ref[...]
    valid = N - i * TC_NB
    node_pos = lax.broadcasted_iota(jnp.int32, (B, F_IN, TC_NB), 2)
    xb = jnp.where(node_pos < valid, xb, jnp.float32(0.0))
    for bb in range(B):
        acc_ref[bb] += lax.dot_general(
            vb, xb[bb], (((1,), (1,)), ((), ())),
            preferred_element_type=jnp.float32)

    @pl.when(i == TC_STEPS - 1)
    def _epi():
        acc = acc_ref[...]
        h = h_ref[...]
        out = b_ref[...]
        for kk in range(K):
            out = out + jnp.dot(acc[:, kk, :], h[kk],
                                preferred_element_type=jnp.float32)
        out_ref[...] = out


def _tc_reduce(v, xt, h2, b2):
    return pl.pallas_call(
        _tc_reduce_body,
        grid=(TC_STEPS,),
        in_specs=[
            pl.BlockSpec(memory_space=pl.ANY),
            pl.BlockSpec((B, F_IN, TC_NB), lambda i: (0, 0, i)),
            pl.BlockSpec((KROWS, F_IN, F_OUT), lambda i: (0, 0, 0)),
            pl.BlockSpec((1, F_OUT), lambda i: (0, 0)),
        ],
        out_specs=pl.BlockSpec((B, F_OUT), lambda i: (0, 0)),
        out_shape=jax.ShapeDtypeStruct((B, F_OUT), jnp.float32),
        scratch_shapes=[
            pltpu.VMEM((B, KROWS, F_IN), jnp.float32),
            pltpu.VMEM((KROWS, NPAD), jnp.float32),
            pltpu.SemaphoreType.DMA,
        ],
    )(v, xt, h2, b2)


def kernel(x, edge_row, edge_col, edge_w, H, b):
    row_p = edge_row.reshape(EROWS, 128)
    col_p = edge_col.reshape(EROWS, 128)
    w_p = edge_w.reshape(EROWS, 128)

    v = _sc_taps(row_p, col_p, w_p).reshape(KROWS, NPAD)

    xt = jnp.transpose(x, (0, 2, 1))
    h2 = jnp.zeros((KROWS, F_IN, F_OUT), jnp.float32).at[:K].set(H[0])
    return _tc_reduce(v, xt, h2, b.reshape(1, F_OUT))
